# baseline (speedup 1.0000x reference)
import functools

import jax
import jax.numpy as jnp
from jax import lax
from jax.experimental import pallas as pl
from jax.experimental.pallas import tpu as pltpu
from jax.experimental.pallas import tpu_sc as plsc

_BATCH = 4096
_SEQ = 200
_D = 64
_H = 128
_NW = 32
_BPW = _BATCH // _NW
_CHUNK = 100
_NCHUNK = _SEQ // _CHUNK


def _pool_body(x_hbm, table_hbm, out_hbm, idx_v, rows_v, out_v, sem0, sem1):
    wid = lax.axis_index("s") * 2 + lax.axis_index("c")
    pltpu.sync_copy(x_hbm.at[wid], idx_v)

    sems = (sem0, sem1)

    def issue(row, buf, sem):
        c0 = pltpu.async_copy(
            table_hbm.at[idx_v.at[2 * row]],
            rows_v.at[buf, pl.ds(0, _CHUNK)], sem)
        c1 = pltpu.async_copy(
            table_hbm.at[idx_v.at[2 * row + 1]],
            rows_v.at[buf, pl.ds(_CHUNK, _CHUNK)], sem)
        del c0, c1

    def wait(row, buf, sem):
        pltpu.make_async_copy(
            table_hbm.at[idx_v.at[2 * row]],
            rows_v.at[buf, pl.ds(0, _CHUNK)], sem).wait()
        pltpu.make_async_copy(
            table_hbm.at[idx_v.at[2 * row + 1]],
            rows_v.at[buf, pl.ds(_CHUNK, _CHUNK)], sem).wait()

    issue(0, 0, sem0)
    issue(1, 1, sem1)

    neg_inf = jnp.full((16,), -jnp.inf, jnp.float32)

    def process(row, buf, sem):
        wait(row, buf, sem)

        def red(j, acc):
            a0, a1, a2, a3 = acc
            a0 = jnp.maximum(a0, rows_v[buf, j, pl.ds(0, 16)])
            a1 = jnp.maximum(a1, rows_v[buf, j, pl.ds(16, 16)])
            a2 = jnp.maximum(a2, rows_v[buf, j, pl.ds(32, 16)])
            a3 = jnp.maximum(a3, rows_v[buf, j, pl.ds(48, 16)])
            return (a0, a1, a2, a3)

        a0, a1, a2, a3 = lax.fori_loop(
            0, _SEQ, red, (neg_inf, neg_inf, neg_inf, neg_inf), unroll=2)
        out_v[row, pl.ds(0, 16)] = a0
        out_v[row, pl.ds(16, 16)] = a1
        out_v[row, pl.ds(32, 16)] = a2
        out_v[row, pl.ds(48, 16)] = a3

        @pl.when(row + 2 < _BPW)
        def _():
            issue(row + 2, buf, sem)

    def body(g, carry):
        process(2 * g, 0, sems[0])
        process(2 * g + 1, 1, sems[1])
        return carry

    lax.fori_loop(0, _BPW // 2, body, 0)

    pltpu.sync_copy(out_v, out_hbm.at[pl.ds(wid * _BPW, _BPW)])


_pool = functools.partial(
    pl.kernel,
    out_type=jax.ShapeDtypeStruct((_BATCH, _D), jnp.float32),
    mesh=plsc.VectorSubcoreMesh(core_axis_name="c", subcore_axis_name="s"),
    scratch_types=[
        pltpu.VMEM((_BPW * _NCHUNK, _CHUNK), jnp.int32),
        pltpu.VMEM((2, _SEQ, _D), jnp.float32),
        pltpu.VMEM((_BPW, _D), jnp.float32),
        pltpu.SemaphoreType.DMA,
        pltpu.SemaphoreType.DMA,
    ],
    compiler_params=pltpu.CompilerParams(use_tc_tiling_on_sc=False),
)(_pool_body)


def _linear_norm_body(p_ref, wt_ref, b_ref, o_ref):
    h = jnp.dot(p_ref[...], wt_ref[...],
                preferred_element_type=jnp.float32) + b_ref[...]
    nrm = jnp.sqrt(jnp.sum(h * h, axis=1, keepdims=True))
    o_ref[...] = h / jnp.maximum(nrm, 1e-12)


def kernel(x, embed_table, W, b):
    x32 = x.astype(jnp.int32).reshape(_NW, _BPW * _NCHUNK, _CHUNK)
    pooled = _pool(x32, embed_table)

    grid = 8
    blk = _BATCH // grid
    out = pl.pallas_call(
        _linear_norm_body,
        out_shape=jax.ShapeDtypeStruct((_BATCH, _H), jnp.float32),
        grid=(grid,),
        in_specs=[
            pl.BlockSpec((blk, _D), lambda i: (i, 0)),
            pl.BlockSpec((_D, _H), lambda i: (0, 0)),
            pl.BlockSpec((1, _H), lambda i: (0, 0)),
        ],
        out_specs=pl.BlockSpec((blk, _H), lambda i: (i, 0)),
    )(pooled, W.T, b[None, :])
    return out
